# Initial kernel scaffold; baseline (speedup 1.0000x reference)
import functools

import jax
import jax.numpy as jnp
from jax import lax
from jax.experimental import pallas as pl
from jax.experimental.pallas import tpu as pltpu
from jax.experimental.pallas import tpu_sc as plsc

N = 10000
E = 320000
D = 128
R = 200

NC = 2
NS = 16
NW = NC * NS
EPW = E // NW
C = 80
CHUNKS = EPW // C
ROWS_PER_TILE = N // NS


def _precompute_body(h_ref, pos_ref, w1_ref, w2_ref, b_ref, wn_ref, rel_ref,
                     t_ref, rel2_ref):
    hp = jnp.dot(h_ref[...], w1_ref[...], preferred_element_type=jnp.float32)
    hp = hp + jnp.dot(pos_ref[...], w2_ref[...],
                      preferred_element_type=jnp.float32)
    hp = hp + b_ref[...]
    t_ref[...] = jnp.dot(hp, wn_ref[...], preferred_element_type=jnp.float32)
    rel2_ref[...] = jnp.dot(rel_ref[...], wn_ref[...],
                            preferred_element_type=jnp.float32)


def _precompute(h, pos_pad, w1, w2_pad, b, wn, emb_rel):
    return pl.pallas_call(
        _precompute_body,
        out_shape=(
            jax.ShapeDtypeStruct((N, D), jnp.float32),
            jax.ShapeDtypeStruct((R, D), jnp.float32),
        ),
    )(h, pos_pad, w1, w2_pad, b, wn, emb_rel)


def _edge_body(t_hbm, rel2_hbm, src_hbm, dst_hbm, et_hbm, zeros_hbm, out_hbm,
               src_v, dst_v, et_v, trow_v, rrow_v, acc, sem1, sem2):
    cid = lax.axis_index("c")
    sid = lax.axis_index("s")
    wid = sid * NC + cid

    row0 = sid * ROWS_PER_TILE
    pltpu.sync_copy(zeros_hbm.at[pl.ds(row0, ROWS_PER_TILE)],
                    acc.at[pl.ds(row0, ROWS_PER_TILE)])

    pltpu.sync_copy(src_hbm.at[wid], src_v)
    pltpu.sync_copy(dst_hbm.at[wid], dst_v)
    pltpu.sync_copy(et_hbm.at[wid], et_v)
    plsc.subcore_barrier()

    def chunk(j, carry):
        cp1 = pltpu.async_copy(t_hbm.at[src_v.at[j]], trow_v, sem1)
        cp2 = pltpu.async_copy(rel2_hbm.at[et_v.at[j]], rrow_v, sem2)
        cp1.wait()
        cp2.wait()
        pltpu.sync_copy(trow_v, acc.at[dst_v.at[j]], add=True)
        pltpu.sync_copy(rrow_v, acc.at[dst_v.at[j]], add=True)
        return carry

    lax.fori_loop(0, CHUNKS, chunk, 0)
    plsc.subcore_barrier()

    pltpu.sync_copy(acc.at[pl.ds(row0, ROWS_PER_TILE)],
                    out_hbm.at[cid, pl.ds(row0, ROWS_PER_TILE)])


_edge_kernel = functools.partial(
    pl.kernel,
    out_type=jax.ShapeDtypeStruct((NC, N, D), jnp.float32),
    mesh=plsc.VectorSubcoreMesh(core_axis_name="c", subcore_axis_name="s",
                                num_cores=NC, num_subcores=NS),
    scratch_types=[
        pltpu.VMEM((CHUNKS, C), jnp.int32),
        pltpu.VMEM((CHUNKS, C), jnp.int32),
        pltpu.VMEM((CHUNKS, C), jnp.int32),
        pltpu.VMEM((C, D), jnp.float32),
        pltpu.VMEM((C, D), jnp.float32),
        pltpu.VMEM_SHARED((N, D), jnp.float32),
        pltpu.SemaphoreType.DMA,
        pltpu.SemaphoreType.DMA,
    ],
)(_edge_body)


def _combine_body(p_ref, norm_ref, o_ref):
    o_ref[...] = (p_ref[0] + p_ref[1]) * norm_ref[...]


def _combine(partials, norm):
    return pl.pallas_call(
        _combine_body,
        out_shape=jax.ShapeDtypeStruct((N, D), jnp.float32),
    )(partials, norm)


def kernel(h, pos_enc, norm, prev_h, emb_rel, W_hp, b_hp, W_neighbor,
           edge_index, edge_type):
    w1 = W_hp[:D]
    w2_pad = jnp.zeros((8, D), jnp.float32).at[:3].set(W_hp[D:])
    pos_pad = jnp.zeros((N, 8), jnp.float32).at[:, :3].set(pos_enc)
    b = jnp.broadcast_to(b_hp[None, :], (1, D))

    t, rel2 = _precompute(h, pos_pad, w1, w2_pad, b, W_neighbor, emb_rel)

    src = edge_index[0].reshape(NW, CHUNKS, C)
    dst = edge_index[1].reshape(NW, CHUNKS, C)
    et = edge_type.reshape(NW, CHUNKS, C)
    zeros = jnp.zeros((N, D), jnp.float32)

    partials = _edge_kernel(t, rel2, src, dst, et, zeros)
    node_repr = _combine(partials, norm)
    return node_repr, pos_enc


# speedup vs baseline: 6.5802x; 6.5802x over previous
import functools

import jax
import jax.numpy as jnp
from jax import lax
from jax.experimental import pallas as pl
from jax.experimental.pallas import tpu as pltpu
from jax.experimental.pallas import tpu_sc as plsc

N = 10000
E = 320000
D = 128
R = 200

NC = 2
NS = 16
NW = NC * NS
EPW = E // NW
C = 80
CHUNKS = EPW // C
NPAD = 10240
ROWS_PER_TILE = NPAD // NS


def _precompute_body(h_ref, pos_ref, w1_ref, w2_ref, b_ref, wn_ref, rel_ref,
                     t_ref, rel2_ref):
    hp = jnp.dot(h_ref[...], w1_ref[...], preferred_element_type=jnp.float32)
    hp = hp + jnp.dot(pos_ref[...], w2_ref[...],
                      preferred_element_type=jnp.float32)
    hp = hp + b_ref[...]
    t_ref[...] = jnp.dot(hp, wn_ref[...], preferred_element_type=jnp.float32)
    rel2_ref[...] = jnp.dot(rel_ref[...], wn_ref[...],
                            preferred_element_type=jnp.float32)


def _precompute(h, pos_pad, w1, w2_pad, b, wn, emb_rel):
    return pl.pallas_call(
        _precompute_body,
        out_shape=(
            jax.ShapeDtypeStruct((N, D), jnp.float32),
            jax.ShapeDtypeStruct((R, D), jnp.float32),
        ),
    )(h, pos_pad, w1, w2_pad, b, wn, emb_rel)


def _edge_body(t_hbm, rel2_hbm, src_hbm, dst_hbm, et_hbm, zeros_hbm, out_hbm,
               src_c, dst_c, et_c, trow_v, rrow_v, acc, sem1, sem2):
    cid = lax.axis_index("c")
    sid = lax.axis_index("s")
    wid = sid * NC + cid

    row0 = sid * ROWS_PER_TILE
    pltpu.sync_copy(zeros_hbm.at[pl.ds(row0, ROWS_PER_TILE)],
                    acc.at[pl.ds(row0, ROWS_PER_TILE)])
    plsc.subcore_barrier()

    def chunk(j, carry):
        pltpu.sync_copy(src_hbm.at[wid, j], src_c)
        pltpu.sync_copy(dst_hbm.at[wid, j], dst_c)
        pltpu.sync_copy(et_hbm.at[wid, j], et_c)
        cp1 = pltpu.async_copy(t_hbm.at[src_c], trow_v, sem1)
        cp2 = pltpu.async_copy(rel2_hbm.at[et_c], rrow_v, sem2)
        cp1.wait()
        cp2.wait()
        pltpu.sync_copy(trow_v, acc.at[dst_c], add=True)
        pltpu.sync_copy(rrow_v, acc.at[dst_c], add=True)
        return carry

    lax.fori_loop(0, CHUNKS, chunk, 0)
    plsc.subcore_barrier()

    pltpu.sync_copy(acc.at[pl.ds(row0, ROWS_PER_TILE)],
                    out_hbm.at[cid, pl.ds(row0, ROWS_PER_TILE)])


_edge_kernel = functools.partial(
    pl.kernel,
    out_type=jax.ShapeDtypeStruct((NC, NPAD, D), jnp.float32),
    mesh=plsc.VectorSubcoreMesh(core_axis_name="c", subcore_axis_name="s",
                                num_cores=NC, num_subcores=NS),
    scratch_types=[
        pltpu.VMEM((C,), jnp.int32),
        pltpu.VMEM((C,), jnp.int32),
        pltpu.VMEM((C,), jnp.int32),
        pltpu.VMEM((C, D), jnp.float32),
        pltpu.VMEM((C, D), jnp.float32),
        pltpu.VMEM_SHARED((NPAD, D), jnp.float32),
        pltpu.SemaphoreType.DMA,
        pltpu.SemaphoreType.DMA,
    ],
)(_edge_body)


def _combine_body(p_ref, norm_ref, o_ref):
    o_ref[...] = (p_ref[0, :N] + p_ref[1, :N]) * norm_ref[...]


def _combine(partials, norm):
    return pl.pallas_call(
        _combine_body,
        out_shape=jax.ShapeDtypeStruct((N, D), jnp.float32),
    )(partials, norm)


def kernel(h, pos_enc, norm, prev_h, emb_rel, W_hp, b_hp, W_neighbor,
           edge_index, edge_type):
    w1 = W_hp[:D]
    w2_pad = jnp.zeros((8, D), jnp.float32).at[:3].set(W_hp[D:])
    pos_pad = jnp.zeros((N, 8), jnp.float32).at[:, :3].set(pos_enc)
    b = jnp.broadcast_to(b_hp[None, :], (1, D))

    t, rel2 = _precompute(h, pos_pad, w1, w2_pad, b, W_neighbor, emb_rel)

    src = edge_index[0].reshape(NW, CHUNKS, C)
    dst = edge_index[1].reshape(NW, CHUNKS, C)
    et = edge_type.reshape(NW, CHUNKS, C)
    zeros = jnp.zeros((NPAD, D), jnp.float32)

    partials = _edge_kernel(t, rel2, src, dst, et, zeros)
    node_repr = _combine(partials, norm)
    return node_repr, pos_enc


# speedup vs baseline: 11.1011x; 1.6871x over previous
import functools

import jax
import jax.numpy as jnp
from jax import lax
from jax.experimental import pallas as pl
from jax.experimental.pallas import tpu as pltpu
from jax.experimental.pallas import tpu_sc as plsc

N = 10000
E = 320000
D = 128
R = 200

NC = 2
NS = 16
NW = NC * NS
EPW = E // NW
C = 80
CHUNKS = EPW // C
NPAD = 10240
ROWS_PER_TILE = NPAD // NS


def _precompute_body(h_ref, pos_ref, w1_ref, w2_ref, b_ref, wn_ref, rel_ref,
                     t_ref, rel2_ref):
    hp = jnp.dot(h_ref[...], w1_ref[...], preferred_element_type=jnp.float32)
    hp = hp + jnp.dot(pos_ref[...], w2_ref[...],
                      preferred_element_type=jnp.float32)
    hp = hp + b_ref[...]
    t_ref[...] = jnp.dot(hp, wn_ref[...], preferred_element_type=jnp.float32)
    rel2_ref[...] = jnp.dot(rel_ref[...], wn_ref[...],
                            preferred_element_type=jnp.float32)


def _precompute(h, pos_pad, w1, w2_pad, b, wn, emb_rel):
    return pl.pallas_call(
        _precompute_body,
        out_shape=(
            jax.ShapeDtypeStruct((N, D), jnp.float32),
            jax.ShapeDtypeStruct((R, D), jnp.float32),
        ),
    )(h, pos_pad, w1, w2_pad, b, wn, emb_rel)


def _edge_body(t_hbm, rel2_hbm, src_hbm, dst_hbm, et_hbm, zeros_hbm, out_hbm,
               src_c, dst_c, et_c, trow_v, rrow_v, acc,
               isem, gsem, ssem):
    cid = lax.axis_index("c")
    sid = lax.axis_index("s")
    wid = sid * NC + cid

    row0 = sid * ROWS_PER_TILE
    pltpu.sync_copy(zeros_hbm.at[pl.ds(row0, ROWS_PER_TILE)],
                    acc.at[pl.ds(row0, ROWS_PER_TILE)])
    plsc.subcore_barrier()

    def stage_a(j, b):
        e0 = wid * EPW + j * C
        pltpu.async_copy(src_hbm.at[pl.ds(e0, C)], src_c[b], isem[b])
        pltpu.async_copy(dst_hbm.at[pl.ds(e0, C)], dst_c[b], isem[b])
        pltpu.async_copy(et_hbm.at[pl.ds(e0, C)], et_c[b], isem[b])

    def stage_b(j, b):
        e0 = wid * EPW + j * C
        pltpu.make_async_copy(src_hbm.at[pl.ds(e0, C)], src_c[b], isem[b]).wait()
        pltpu.make_async_copy(dst_hbm.at[pl.ds(e0, C)], dst_c[b], isem[b]).wait()
        pltpu.make_async_copy(et_hbm.at[pl.ds(e0, C)], et_c[b], isem[b]).wait()
        pltpu.async_copy(t_hbm.at[src_c[b]], trow_v[b], gsem[b])
        pltpu.async_copy(rel2_hbm.at[et_c[b]], rrow_v[b], gsem[b])

    def wait_b(b):
        pltpu.make_async_copy(t_hbm.at[src_c[b]], trow_v[b], gsem[b]).wait()
        pltpu.make_async_copy(rel2_hbm.at[et_c[b]], rrow_v[b], gsem[b]).wait()

    def stage_c(b):
        wait_b(b)
        pltpu.async_copy(trow_v[b], acc.at[dst_c[b]], ssem[b], add=True)
        pltpu.async_copy(rrow_v[b], acc.at[dst_c[b]], ssem[b], add=True)

    def wait_c(b):
        pltpu.make_async_copy(trow_v[b], acc.at[dst_c[b]], ssem[b]).wait()
        pltpu.make_async_copy(rrow_v[b], acc.at[dst_c[b]], ssem[b]).wait()

    stage_a(0, 0)
    stage_a(1, 1)
    stage_b(0, 0)
    stage_b(1, 1)

    def steady(g, carry):
        for b in range(2):
            j = 2 * g + b
            stage_c(b)
            wait_c(b)
            stage_a(j, b)
            stage_b(j, b)
        return carry

    lax.fori_loop(1, (CHUNKS - 1) // 2, steady, 0)
    stage_c(0)
    wait_c(0)
    stage_a(CHUNKS - 1, 0)
    stage_b(CHUNKS - 1, 0)
    stage_c(1)
    wait_c(1)
    stage_c(0)
    wait_c(0)

    plsc.subcore_barrier()
    pltpu.sync_copy(acc.at[pl.ds(row0, ROWS_PER_TILE)],
                    out_hbm.at[cid, pl.ds(row0, ROWS_PER_TILE)])


_edge_kernel = functools.partial(
    pl.kernel,
    out_type=jax.ShapeDtypeStruct((NC, NPAD, D), jnp.float32),
    mesh=plsc.VectorSubcoreMesh(core_axis_name="c", subcore_axis_name="s",
                                num_cores=NC, num_subcores=NS),
    scratch_types=[
        [pltpu.VMEM((C,), jnp.int32)] * 2,
        [pltpu.VMEM((C,), jnp.int32)] * 2,
        [pltpu.VMEM((C,), jnp.int32)] * 2,
        [pltpu.VMEM((C, D), jnp.float32)] * 2,
        [pltpu.VMEM((C, D), jnp.float32)] * 2,
        pltpu.VMEM_SHARED((NPAD, D), jnp.float32),
        [pltpu.SemaphoreType.DMA] * 2,
        [pltpu.SemaphoreType.DMA] * 2,
        [pltpu.SemaphoreType.DMA] * 2,
    ],
)(_edge_body)


def _combine_body(p_ref, norm_ref, o_ref):
    o_ref[...] = (p_ref[0, :N] + p_ref[1, :N]) * norm_ref[...]


def _combine(partials, norm):
    return pl.pallas_call(
        _combine_body,
        out_shape=jax.ShapeDtypeStruct((N, D), jnp.float32),
    )(partials, norm)


def kernel(h, pos_enc, norm, prev_h, emb_rel, W_hp, b_hp, W_neighbor,
           edge_index, edge_type):
    w1 = W_hp[:D]
    w2_pad = jnp.zeros((8, D), jnp.float32).at[:3].set(W_hp[D:])
    pos_pad = jnp.zeros((N, 8), jnp.float32).at[:, :3].set(pos_enc)
    b = jnp.broadcast_to(b_hp[None, :], (1, D))

    t, rel2 = _precompute(h, pos_pad, w1, w2_pad, b, W_neighbor, emb_rel)

    src = edge_index[0]
    dst = edge_index[1]
    et = edge_type
    zeros = jnp.zeros((NPAD, D), jnp.float32)

    partials = _edge_kernel(t, rel2, src, dst, et, zeros)
    node_repr = _combine(partials, norm)
    return node_repr, pos_enc
